# Initial kernel scaffold; baseline (speedup 1.0000x reference)
import jax
import jax.numpy as jnp
from jax.experimental import pallas as pl


def kernel(x, edge_index, W1, b1, W2, b2):
    raise NotImplementedError("write your pallas kernel here")


# speedup vs baseline: 34.1672x; 34.1672x over previous
import functools

import jax
import jax.numpy as jnp
from jax import lax
from jax.experimental import pallas as pl
from jax.experimental.pallas import tpu as pltpu
from jax.experimental.pallas import tpu_sc as plsc

NC = 2
NS = 16
NW = NC * NS
CHUNK = 128


def _deg_sc(np_, nchunk):
    mesh = plsc.VectorSubcoreMesh(core_axis_name="c", subcore_axis_name="s")
    seg = np_ // NS

    @functools.partial(
        pl.kernel,
        out_type=jax.ShapeDtypeStruct((NC, np_), jnp.float32),
        mesh=mesh,
        scratch_types=[
            pltpu.VMEM_SHARED((np_,), jnp.float32),
            pltpu.VMEM((seg,), jnp.float32),
            pltpu.VMEM((CHUNK,), jnp.float32),
            pltpu.VMEM((CHUNK,), jnp.int32),
        ],
    )
    def deg_kernel(dst_hbm, out_hbm, deg_sh, seg_v, ones_v, idx_v):
        c = lax.axis_index("c")
        s = lax.axis_index("s")

        def zbody(i, _):
            seg_v[pl.ds(i * 16, 16)] = jnp.zeros((16,), jnp.float32)
            return 0

        lax.fori_loop(0, seg // 16, zbody, 0)
        for i in range(CHUNK // 16):
            ones_v[pl.ds(i * 16, 16)] = jnp.ones((16,), jnp.float32)
        base = s * seg
        pltpu.sync_copy(seg_v, deg_sh.at[pl.ds(base, seg)])
        plsc.subcore_barrier()

        w = s * NC + c
        lo = w * nchunk // NW
        hi = (w + 1) * nchunk // NW

        def body(j, _):
            pltpu.sync_copy(dst_hbm.at[j], idx_v)
            pltpu.sync_copy(ones_v, deg_sh.at[idx_v], add=True)
            return 0

        lax.fori_loop(lo, hi, body, 0)
        plsc.subcore_barrier()
        pltpu.sync_copy(deg_sh.at[pl.ds(base, seg)], seg_v)
        pltpu.sync_copy(seg_v, out_hbm.at[c, pl.ds(base, seg)])

    return deg_kernel


def _scatter_sc(np_, dh, nchunk):
    mesh = plsc.VectorSubcoreMesh(core_axis_name="c", subcore_axis_name="s")
    seg = np_ // NS

    @functools.partial(
        pl.kernel,
        out_type=jax.ShapeDtypeStruct((NC, np_, dh), jnp.float32),
        mesh=mesh,
        scratch_types=[
            pltpu.VMEM_SHARED((np_, dh), jnp.float32),
            pltpu.VMEM_SHARED((np_, dh), jnp.float32),
            pltpu.VMEM((seg, dh), jnp.float32),
            pltpu.VMEM((CHUNK,), jnp.int32),
            pltpu.VMEM((CHUNK,), jnp.int32),
            pltpu.VMEM((CHUNK, dh), jnp.float32),
        ],
    )
    def scatter_kernel(src_hbm, dst_hbm, hs_hbm, out_hbm,
                       hs_sh, agg_sh, stage_v, si_v, di_v, rows_v):
        c = lax.axis_index("c")
        s = lax.axis_index("s")
        base = s * seg
        pltpu.sync_copy(hs_hbm.at[pl.ds(base, seg)], stage_v)
        pltpu.sync_copy(stage_v, hs_sh.at[pl.ds(base, seg)])
        pltpu.sync_copy(stage_v, agg_sh.at[pl.ds(base, seg)])
        plsc.subcore_barrier()

        w = s * NC + c
        lo = w * nchunk // NW
        hi = (w + 1) * nchunk // NW

        def body(j, _):
            pltpu.sync_copy(src_hbm.at[j], si_v)
            pltpu.sync_copy(dst_hbm.at[j], di_v)
            pltpu.sync_copy(hs_sh.at[si_v], rows_v)
            pltpu.sync_copy(rows_v, agg_sh.at[di_v], add=True)
            return 0

        lax.fori_loop(lo, hi, body, 0)
        plsc.subcore_barrier()
        pltpu.sync_copy(agg_sh.at[pl.ds(base, seg)], stage_v)
        pltpu.sync_copy(stage_v, out_hbm.at[c, pl.ds(base, seg)])

    return scatter_kernel


def _hs_body(x_ref, w1_ref, degt_ref, hs_ref):
    h = jnp.dot(x_ref[...], w1_ref[...], preferred_element_type=jnp.float32)
    degsum = degt_ref[:, 0:1] + degt_ref[:, 1:2] + 1.0
    hs_ref[...] = h * lax.rsqrt(degsum)


def _out_body(agg_ref, hs_ref, degt_ref, w2_ref, b1_ref, b2_ref, o_ref):
    degsum = degt_ref[:, 0:1] + degt_ref[:, 1:2] + 1.0
    dis = lax.rsqrt(degsum)
    conv = dis * (agg_ref[0] + agg_ref[1] - hs_ref[...]) + b1_ref[...]
    o_ref[...] = (
        jnp.dot(conv, w2_ref[...], preferred_element_type=jnp.float32)
        + b2_ref[...]
    )


def kernel(x, edge_index, W1, b1, W2, b2):
    n, d_in = x.shape
    e = edge_index.shape[1]
    d_hid = W1.shape[1]
    d_out = W2.shape[1]
    np_ = ((n + 2047) // 2048) * 2048
    assert e % CHUNK == 0
    nchunk = e // CHUNK

    ei = edge_index.astype(jnp.int32)
    src2d = ei[0].reshape(nchunk, CHUNK)
    dst2d = ei[1].reshape(nchunk, CHUNK)
    x_pad = jnp.pad(x, ((0, np_ - n), (0, 0)))

    deg = _deg_sc(np_, nchunk)(dst2d)
    degt = deg.T

    hs = pl.pallas_call(
        _hs_body,
        out_shape=jax.ShapeDtypeStruct((np_, d_hid), jnp.float32),
    )(x_pad, W1, degt)

    agg = _scatter_sc(np_, d_hid, nchunk)(src2d, dst2d, hs)

    out = pl.pallas_call(
        _out_body,
        out_shape=jax.ShapeDtypeStruct((np_, d_out), jnp.float32),
    )(agg, hs, degt, W2, b1.reshape(1, d_hid), b2.reshape(1, d_out))

    return out[:n]
